# baseline (speedup 1.0000x reference)
import jax
import jax.numpy as jnp
from jax import lax
from jax.experimental import pallas as pl
from jax.experimental.pallas import tpu as pltpu
from jax.experimental.pallas import tpu_sc as plsc

BATCH = 16384
LATENT = 64
NC = 2
NS = 16
NW = NC * NS
BPW = BATCH // NW
HALF = BPW // 2
CHUNK = 128
NCH = HALF // CHUNK


def _body(sh_idx, raw_idx, inv_tab, stk_tab,
          out_pos, out_neg,
          raw_i, raw_p, raw_n,
          shi_i, shi_p, shi_n,
          inv_rows, pos_rows, neg_rows,
          outp_v, outn_v, sem):
    wid = lax.axis_index("s") * NC + lax.axis_index("c")
    base = wid * BPW
    lanes = lax.broadcasted_iota(jnp.int32, (16,), 0)

    for h in range(2):
        pltpu.sync_copy(raw_idx.at[0].at[wid].at[h], raw_i)
        pltpu.sync_copy(raw_idx.at[1].at[wid].at[h], raw_p)
        pltpu.sync_copy(raw_idx.at[2].at[wid].at[h], raw_n)
        pltpu.sync_copy(sh_idx.at[0].at[wid].at[h], shi_i)
        pltpu.sync_copy(sh_idx.at[1].at[wid].at[h], shi_p)
        pltpu.sync_copy(sh_idx.at[2].at[wid].at[h], shi_n)

        copies = []
        for j in range(NCH):
            dst = pl.ds(j * CHUNK, CHUNK)
            copies.append(pltpu.async_copy(
                inv_tab.at[shi_i.at[j]], inv_rows.at[dst], sem))
            copies.append(pltpu.async_copy(
                stk_tab.at[shi_p.at[j]], pos_rows.at[dst], sem))
            copies.append(pltpu.async_copy(
                stk_tab.at[shi_n.at[j]], neg_rows.at[dst], sem))
        for c in copies:
            c.wait()

        @plsc.parallel_loop(0, HALF // 16)
        def group(g):
            rows16 = g * 16 + lanes
            sl16 = pl.ds(g * 16, 16)
            pi = (raw_i[sl16] & 1) * LATENT
            pp = (raw_p[sl16] & 1) * LATENT
            pn = (raw_n[sl16] & 1) * LATENT
            acc = [jnp.zeros((16,), jnp.float32) for _ in range(8)]
            for d in range(LATENT):
                a = plsc.load_gather(inv_rows, [rows16, pi + d])
                p = plsc.load_gather(pos_rows, [rows16, pp + d])
                n = plsc.load_gather(neg_rows, [rows16, pn + d])
                k = d % 4
                acc[k] = acc[k] + a * p
                acc[4 + k] = acc[4 + k] + a * n
            outp_v[pl.ds(h * HALF + g * 16, 16)] = (acc[0] + acc[1]) + (acc[2] + acc[3])
            outn_v[pl.ds(h * HALF + g * 16, 16)] = (acc[4] + acc[5]) + (acc[6] + acc[7])

    pltpu.sync_copy(outp_v, out_pos.at[pl.ds(base, BPW)])
    pltpu.sync_copy(outn_v, out_neg.at[pl.ds(base, BPW)])


_mf_bpr = pl.kernel(
    _body,
    out_type=[
        jax.ShapeDtypeStruct((BATCH,), jnp.float32),
        jax.ShapeDtypeStruct((BATCH,), jnp.float32),
    ],
    mesh=plsc.VectorSubcoreMesh(core_axis_name="c", subcore_axis_name="s"),
    compiler_params=pltpu.CompilerParams(
        needs_layout_passes=False, use_tc_tiling_on_sc=True
    ),
    scratch_types=[
        pltpu.VMEM((HALF,), jnp.int32),
        pltpu.VMEM((HALF,), jnp.int32),
        pltpu.VMEM((HALF,), jnp.int32),
        pltpu.VMEM((NCH, CHUNK), jnp.int32),
        pltpu.VMEM((NCH, CHUNK), jnp.int32),
        pltpu.VMEM((NCH, CHUNK), jnp.int32),
        pltpu.VMEM((HALF, 128), jnp.float32),
        pltpu.VMEM((HALF, 128), jnp.float32),
        pltpu.VMEM((HALF, 128), jnp.float32),
        pltpu.VMEM((BPW,), jnp.float32),
        pltpu.VMEM((BPW,), jnp.float32),
        pltpu.SemaphoreType.DMA,
    ],
)


@jax.jit
def kernel(investor, stock_positive, stock_negative, embed_investor, embed_stock):
    idx = jnp.stack([
        investor.astype(jnp.int32),
        stock_positive.astype(jnp.int32),
        stock_negative.astype(jnp.int32),
    ])
    raw = idx.reshape(3, NW, 2, HALF)
    shifted = (idx >> 1).reshape(3, NW, 2, NCH, CHUNK)
    inv_pairs = embed_investor.reshape(-1, 2 * LATENT)
    stk_pairs = embed_stock.reshape(-1, 2 * LATENT)
    out_p, out_n = _mf_bpr(shifted, raw, inv_pairs, stk_pairs)
    return (out_p, out_n)


# speedup vs baseline: 1.0183x; 1.0183x over previous
import jax
import jax.numpy as jnp
from jax import lax
from jax.experimental import pallas as pl
from jax.experimental.pallas import tpu as pltpu
from jax.experimental.pallas import tpu_sc as plsc

BATCH = 16384
LATENT = 64
NC = 2
NS = 16
NW = NC * NS
BPW = BATCH // NW
CHUNK = 128
NCHUNK = BPW // CHUNK


def _body(inv_idx, pos_idx, neg_idx, inv_tab, stk_tab,
          out_pos, out_neg,
          idx_inv, idx_pos, idx_neg,
          inv_rows, pos_rows, neg_rows,
          outp_v, outn_v, sem):
    wid = lax.axis_index("s") * NC + lax.axis_index("c")
    base = wid * BPW

    pltpu.sync_copy(inv_idx.at[wid], idx_inv)
    pltpu.sync_copy(pos_idx.at[wid], idx_pos)
    pltpu.sync_copy(neg_idx.at[wid], idx_neg)

    copies = []
    for j in range(NCHUNK):
        dst = pl.ds(j * CHUNK, CHUNK)
        copies.append(pltpu.async_copy(inv_tab.at[idx_inv.at[j]], inv_rows.at[dst], sem))
        copies.append(pltpu.async_copy(stk_tab.at[idx_pos.at[j]], pos_rows.at[dst], sem))
        copies.append(pltpu.async_copy(stk_tab.at[idx_neg.at[j]], neg_rows.at[dst], sem))
    for c in copies:
        c.wait()

    lanes = lax.broadcasted_iota(jnp.int32, (16,), 0)

    @plsc.parallel_loop(0, BPW // 16)
    def group(g):
        rows16 = g * 16 + lanes
        acc = [jnp.zeros((16,), jnp.float32) for _ in range(8)]
        for d in range(LATENT):
            dcol = jnp.full((16,), d, jnp.int32)
            a = plsc.load_gather(inv_rows, [rows16, dcol])
            p = plsc.load_gather(pos_rows, [rows16, dcol])
            n = plsc.load_gather(neg_rows, [rows16, dcol])
            k = d % 4
            acc[k] = acc[k] + a * p
            acc[4 + k] = acc[4 + k] + a * n
        outp_v[pl.ds(g * 16, 16)] = (acc[0] + acc[1]) + (acc[2] + acc[3])
        outn_v[pl.ds(g * 16, 16)] = (acc[4] + acc[5]) + (acc[6] + acc[7])

    pltpu.sync_copy(outp_v, out_pos.at[pl.ds(base, BPW)])
    pltpu.sync_copy(outn_v, out_neg.at[pl.ds(base, BPW)])


_mf_bpr = pl.kernel(
    _body,
    out_type=[
        jax.ShapeDtypeStruct((BATCH,), jnp.float32),
        jax.ShapeDtypeStruct((BATCH,), jnp.float32),
    ],
    mesh=plsc.VectorSubcoreMesh(core_axis_name="c", subcore_axis_name="s"),
    compiler_params=pltpu.CompilerParams(
        needs_layout_passes=False, use_tc_tiling_on_sc=False
    ),
    scratch_types=[
        pltpu.VMEM((NCHUNK, CHUNK), jnp.int32),
        pltpu.VMEM((NCHUNK, CHUNK), jnp.int32),
        pltpu.VMEM((NCHUNK, CHUNK), jnp.int32),
        pltpu.VMEM((BPW, LATENT), jnp.float32),
        pltpu.VMEM((BPW, LATENT), jnp.float32),
        pltpu.VMEM((BPW, LATENT), jnp.float32),
        pltpu.VMEM((BPW,), jnp.float32),
        pltpu.VMEM((BPW,), jnp.float32),
        pltpu.SemaphoreType.DMA,
    ],
)


@jax.jit
def kernel(investor, stock_positive, stock_negative, embed_investor, embed_stock):
    inv_idx = investor.astype(jnp.int32).reshape(NW, NCHUNK, CHUNK)
    pos_idx = stock_positive.astype(jnp.int32).reshape(NW, NCHUNK, CHUNK)
    neg_idx = stock_negative.astype(jnp.int32).reshape(NW, NCHUNK, CHUNK)
    out_p, out_n = _mf_bpr(inv_idx, pos_idx, neg_idx, embed_investor, embed_stock)
    return (out_p, out_n)
